# baseline (speedup 1.0000x reference)
import jax
import jax.numpy as jnp
from jax import lax
from jax.experimental import pallas as pl
from jax.experimental.pallas import tpu as pltpu

_B = 1024
_D = 128
_H = 16
_W = 16
_HW = _H * _W
_BB = 16
_NBLK = _B // _BB


def _body(labels_ref, table_ref, out_ref, rows_ref, sems_ref):
    i = pl.program_id(0)
    slot = lax.rem(i, 2)
    nxt = lax.rem(i + 1, 2)

    @pl.when(i == 0)
    def _prologue():
        for j in range(_BB):
            pltpu.make_async_copy(
                table_ref.at[labels_ref[j]], rows_ref.at[0, j], sems_ref.at[0]
            ).start()

    @pl.when(i + 1 < _NBLK)
    def _prefetch():
        for j in range(_BB):
            pltpu.make_async_copy(
                table_ref.at[labels_ref[(i + 1) * _BB + j]],
                rows_ref.at[nxt, j],
                sems_ref.at[nxt],
            ).start()

    for j in range(_BB):
        pltpu.make_async_copy(
            table_ref.at[0], rows_ref.at[slot, j], sems_ref.at[slot]
        ).wait()

    g = rows_ref[slot]
    out_ref[...] = jnp.broadcast_to(g[:, None, :], (_BB, _HW, _D))


def kernel(class_labels, embedding_table):
    labels = class_labels.astype(jnp.int32)
    grid_spec = pltpu.PrefetchScalarGridSpec(
        num_scalar_prefetch=1,
        grid=(_NBLK,),
        in_specs=[pl.BlockSpec(memory_space=pltpu.MemorySpace.HBM)],
        out_specs=pl.BlockSpec((_BB, _HW, _D), lambda i, labels: (i, 0, 0)),
        scratch_shapes=[
            pltpu.VMEM((2, _BB, _D), jnp.float32),
            pltpu.SemaphoreType.DMA((2,)),
        ],
    )
    out2 = pl.pallas_call(
        _body,
        grid_spec=grid_spec,
        out_shape=jax.ShapeDtypeStruct((_B, _HW, _D), jnp.float32),
    )(labels, embedding_table)
    return jnp.transpose(out2.reshape(_B, _H, _W, _D), (0, 3, 1, 2))


# speedup vs baseline: 1.3312x; 1.3312x over previous
import jax
import jax.numpy as jnp
from jax import lax
from jax.experimental import pallas as pl
from jax.experimental.pallas import tpu as pltpu

_B = 1024
_D = 128
_H = 16
_W = 16
_HW = _H * _W
_BB = 32
_NBLK = _B // _BB


def _body(labels_ref, table_ref, out_ref, rows_ref, sems_ref):
    i = pl.program_id(0)
    slot = lax.rem(i, 2)
    nxt = lax.rem(i + 1, 2)

    @pl.when(i == 0)
    def _prologue():
        for j in range(_BB):
            pltpu.make_async_copy(
                table_ref.at[labels_ref[j]], rows_ref.at[0, j], sems_ref.at[0]
            ).start()

    @pl.when(i + 1 < _NBLK)
    def _prefetch():
        for j in range(_BB):
            pltpu.make_async_copy(
                table_ref.at[labels_ref[(i + 1) * _BB + j]],
                rows_ref.at[nxt, j],
                sems_ref.at[nxt],
            ).start()

    for j in range(_BB):
        pltpu.make_async_copy(
            table_ref.at[0], rows_ref.at[slot, j], sems_ref.at[slot]
        ).wait()

    g = rows_ref[slot]
    out_ref[...] = jnp.broadcast_to(g[:, None, :], (_BB, _HW, _D))


def kernel(class_labels, embedding_table):
    labels = class_labels.astype(jnp.int32)
    grid_spec = pltpu.PrefetchScalarGridSpec(
        num_scalar_prefetch=1,
        grid=(_NBLK,),
        in_specs=[pl.BlockSpec(memory_space=pltpu.MemorySpace.HBM)],
        out_specs=pl.BlockSpec((_BB, _HW, _D), lambda i, labels: (i, 0, 0)),
        scratch_shapes=[
            pltpu.VMEM((2, _BB, _D), jnp.float32),
            pltpu.SemaphoreType.DMA((2,)),
        ],
    )
    out2 = pl.pallas_call(
        _body,
        grid_spec=grid_spec,
        out_shape=jax.ShapeDtypeStruct((_B, _HW, _D), jnp.float32),
    )(labels, embedding_table)
    return jnp.transpose(out2.reshape(_B, _H, _W, _D), (0, 3, 1, 2))
